# baseline (speedup 1.0000x reference)
import functools

import jax
import jax.numpy as jnp
import numpy as np
from jax import lax
from jax.experimental import pallas as pl
from jax.experimental.pallas import tpu as pltpu
from jax.experimental.pallas import tpu_sc as plsc

N = 10000
NP = 10240
E = 160000
HID = 256
NH = 8
C = 32
G = 16
L = 8
NF = 10
EF = 4
XC = 288
AEC = 16
F32 = jnp.float32


def _ln(x, g, b):
    m = x.mean(-1, keepdims=True)
    v = ((x - m) ** 2).mean(-1, keepdims=True)
    return (x - m) / jnp.sqrt(v + 1e-5) * g + b


def _embed_body(x_ref, w1_ref, b1_ref, g_ref, bb_ref, w2_ref, b2_ref, o_ref):
    h = jnp.dot(x_ref[...], w1_ref[...], preferred_element_type=F32) + b1_ref[0]
    h = _ln(h, g_ref[0], bb_ref[0])
    h = jax.nn.gelu(h)
    o_ref[...] = jnp.dot(h, w2_ref[...], preferred_element_type=F32) + b2_ref[0]


def _mm_body(h_ref, w_ref, o_ref):
    o_ref[...] = jnp.dot(h_ref[...], w_ref[...], preferred_element_type=F32)


def _combine_body(acc_ref, xa_ref, cnt_ref, hin_ref, b_ref, g_ref, bb_ref, o_ref):
    acc = acc_ref[...]
    xa = xa_ref[...]
    xh = xa[:, :HID]
    a_d = xa[:, HID:HID + NH]
    a_s = xa[:, HID + 16:HID + 16 + NH]
    den8 = acc[:, HID:HID + NH]
    aes8 = acc[:, HID + 2 * NH:HID + 3 * NH]
    cnt = cnt_ref[...]
    ael = aes8 / jnp.maximum(cnt, 1.0)
    z = a_s + a_d + ael
    z = jnp.where(z >= 0, z, 0.2 * z)
    exs = jnp.exp(z)
    heads = lax.broadcasted_iota(jnp.int32, (NH, HID), 1) % NH
    rows = lax.broadcasted_iota(jnp.int32, (NH, HID), 0)
    expm = jnp.where(heads == rows, 1.0, 0.0)
    num = acc[:, :HID] + jnp.dot(exs, expm, preferred_element_type=F32) * xh
    den = jnp.dot(den8 + exs, expm, preferred_element_type=F32)
    h2 = num / den + b_ref[0]
    h2 = jax.nn.gelu(_ln(h2, g_ref[0], bb_ref[0]))
    o_ref[...] = h2 + hin_ref[...]


def _pool_body(h_ref, b_ref, sum_ref, max_ref, cnt_ref):
    i = pl.program_id(0)

    @pl.when(i == 0)
    def _():
        sum_ref[...] = jnp.zeros_like(sum_ref)
        max_ref[...] = jnp.full_like(max_ref, -jnp.inf)
        cnt_ref[...] = jnp.zeros_like(cnt_ref)

    h = h_ref[...]
    bid = b_ref[...]
    bn = h.shape[0]
    gids = lax.broadcasted_iota(jnp.int32, (G, bn), 0)
    onehot = (gids == bid[:, 0][None, :]).astype(F32)
    sum_ref[...] += jnp.dot(onehot, h, preferred_element_type=F32)
    cnt_ref[...] += jnp.dot(onehot, jnp.ones((bn, NH), F32),
                            preferred_element_type=F32)
    rows = [jnp.max(jnp.where(bid == g, h, -jnp.inf), axis=0, keepdims=True)
            for g in range(G)]
    max_ref[...] = jnp.maximum(max_ref[...], jnp.concatenate(rows, axis=0))


def _heads_body(*refs):
    sum_ref, max_ref, cnt_ref = refs[0], refs[1], refs[2]
    wrefs = refs[3:-2]
    o1_ref, o2_ref = refs[-2], refs[-1]
    s = sum_ref[...]
    mx = max_ref[...]
    mx = jnp.where(mx > -1e37, mx, 0.0)
    c = jnp.maximum(cnt_ref[:, :1], 1.0)
    xg = jnp.concatenate([s / c, s, mx], axis=1)

    def head(ws):
        (w1, b1, g1, bb1, w2, b2, g2, bb2, w3, b3, g3, bb3, w4, b4) = ws
        h = xg @ w1[...] + b1[0]
        h = jax.nn.gelu(_ln(h, g1[0], bb1[0]))
        h = h @ w2[...] + b2[0]
        h = jax.nn.gelu(_ln(h, g2[0], bb2[0]))
        h = h @ w3[...] + b3[0]
        h = jax.nn.gelu(_ln(h, g3[0], bb3[0]))
        return jax.nn.sigmoid(h @ w4[...] + b4[0])

    o1_ref[...] = head(wrefs[:14])
    o2_ref[...] = head(wrefs[14:])


def _full(shape):
    return pl.BlockSpec(shape, lambda i: (0,) * len(shape))


def _embed_call(xp, w1, b1, g, bb, w2, b2, bn, out_cols):
    rows = xp.shape[0]
    k = xp.shape[1]
    grid = rows // bn
    return pl.pallas_call(
        _embed_body,
        grid=(grid,),
        in_specs=[
            pl.BlockSpec((bn, k), lambda i: (i, 0)),
            _full((k, HID)), _full((1, HID)), _full((1, HID)), _full((1, HID)),
            _full((HID, out_cols)), _full((1, out_cols)),
        ],
        out_specs=pl.BlockSpec((bn, out_cols), lambda i: (i, 0)),
        out_shape=jax.ShapeDtypeStruct((rows, out_cols), F32),
    )(xp, w1, b1[None], g[None], bb[None], w2, b2[None])


def _mm_call(h, w, bn):
    rows, k = h.shape
    cols = w.shape[1]
    return pl.pallas_call(
        _mm_body,
        grid=(rows // bn,),
        in_specs=[pl.BlockSpec((bn, k), lambda i: (i, 0)), _full((k, cols))],
        out_specs=pl.BlockSpec((bn, cols), lambda i: (i, 0)),
        out_shape=jax.ShapeDtypeStruct((rows, cols), F32),
    )(h, w)


def _combine_call(acc, xa, cntf, hin, b, g, bb, bn):
    grid = NP // bn
    return pl.pallas_call(
        _combine_body,
        grid=(grid,),
        in_specs=[
            pl.BlockSpec((bn, XC), lambda i: (i, 0)),
            pl.BlockSpec((bn, XC), lambda i: (i, 0)),
            pl.BlockSpec((bn, NH), lambda i: (i, 0)),
            pl.BlockSpec((bn, HID), lambda i: (i, 0)),
            _full((1, HID)), _full((1, HID)), _full((1, HID)),
        ],
        out_specs=pl.BlockSpec((bn, HID), lambda i: (i, 0)),
        out_shape=jax.ShapeDtypeStruct((NP, HID), F32),
    )(acc, xa, cntf, hin, b[None], g[None], bb[None])


def _pool_call(h, bid, bn):
    grid = NP // bn
    return pl.pallas_call(
        _pool_body,
        grid=(grid,),
        in_specs=[
            pl.BlockSpec((bn, HID), lambda i: (i, 0)),
            pl.BlockSpec((bn, 1), lambda i: (i, 0)),
        ],
        out_specs=[
            pl.BlockSpec((G, HID), lambda i: (0, 0)),
            pl.BlockSpec((G, HID), lambda i: (0, 0)),
            pl.BlockSpec((G, NH), lambda i: (0, 0)),
        ],
        out_shape=[
            jax.ShapeDtypeStruct((G, HID), F32),
            jax.ShapeDtypeStruct((G, HID), F32),
            jax.ShapeDtypeStruct((G, NH), F32),
        ],
    )(h, bid)


def _heads_call(sums, maxs, cnts, hp1, hp2):
    def wlist(p):
        out = []
        for wk, bk, gk, bbk in (('W1', 'b1', 'ln1_g', 'ln1_b'),
                                ('W2', 'b2', 'ln2_g', 'ln2_b'),
                                ('W3', 'b3', 'ln3_g', 'ln3_b')):
            out += [p[wk], p[bk][None], p[gk][None], p[bbk][None]]
        out += [p['W4'], p['b4'][None]]
        return out

    ws = wlist(hp1) + wlist(hp2)
    specs = [_full(w.shape) for w in ws]
    return pl.pallas_call(
        _heads_body,
        grid=(1,),
        in_specs=[_full((G, HID)), _full((G, HID)), _full((G, NH))] + specs,
        out_specs=[_full((G, 3)), _full((G, 3))],
        out_shape=[jax.ShapeDtypeStruct((G, 3), F32),
                   jax.ShapeDtypeStruct((G, 3), F32)],
    )(sums, maxs, cnts, *ws)


NPW = 320
RPPAD = NPW + 16
CK = 16
SUP = 6
PADE = 320


def _sc_wid():
    return lax.axis_index("s") * 2 + lax.axis_index("c")


def _edge_sc_kernel(xa_hbm, ae_hbm, src_hbm, dst_hbm, rp_hbm, out_hbm,
                    ownad_v, rpe_v, s0, s1, s2, d0, d1, d2, a0, a1, a2,
                    r0, r1, out_v, st0, st1, st2, sr0, sr1):
    wid = _sc_wid()
    n0 = pl.multiple_of(wid * NPW, NPW)

    iota = lax.iota(jnp.int32, 16)
    zero16 = jnp.zeros((16,), F32)

    srcs = (s0, s1, s2)
    dsts = (d0, d1, d2)
    aes = (a0, a1, a2)
    rows = (r0, r1)
    stsems = (st0, st1, st2)
    rsems = (sr0, sr1)

    pltpu.sync_copy(rp_hbm.at[pl.ds(n0, 16)], rpe_v)
    e0 = jnp.sum(jnp.where(iota == 0, rpe_v[...], 0))
    pltpu.sync_copy(rp_hbm.at[pl.ds(n0 + NPW, 16)], rpe_v)
    e1w = jnp.sum(jnp.where(iota == 0, rpe_v[...], 0))
    ebase = pl.multiple_of(e0 - lax.rem(e0, 8), 8)

    pltpu.sync_copy(xa_hbm.at[pl.ds(n0, NPW), pl.ds(HID, 16)],
                    ownad_v.at[pl.ds(0, NPW)])
    ownad_v[NPW, :] = zero16

    def zrow(m, _):
        for k in range(XC // 16):
            out_v[m, pl.ds(16 * k, 16)] = zero16
        return 0
    lax.fori_loop(0, NPW + 1, zrow, 0)

    def stage_issue(c, slot):
        off = ebase + c * CK
        pltpu.make_async_copy(src_hbm.at[pl.ds(off, CK)], srcs[slot],
                              stsems[slot]).start()
        pltpu.make_async_copy(dst_hbm.at[pl.ds(off, CK)], dsts[slot],
                              stsems[slot]).start()
        pltpu.make_async_copy(ae_hbm.at[pl.ds(off, CK), pl.ds(0, AEC)],
                              aes[slot], stsems[slot]).start()

    def stage_wait(c, slot):
        off = ebase + c * CK
        pltpu.make_async_copy(src_hbm.at[pl.ds(off, CK)], srcs[slot],
                              stsems[slot]).wait()
        pltpu.make_async_copy(dst_hbm.at[pl.ds(off, CK)], dsts[slot],
                              stsems[slot]).wait()
        pltpu.make_async_copy(ae_hbm.at[pl.ds(off, CK), pl.ds(0, AEC)],
                              aes[slot], stsems[slot]).wait()

    def rows_issue(rslot, sslot):
        pltpu.make_async_copy(xa_hbm.at[srcs[sslot]], rows[rslot],
                              rsems[rslot]).start()

    def rows_wait(rslot, sslot):
        pltpu.make_async_copy(xa_hbm.at[srcs[sslot]], rows[rslot],
                              rsems[rslot]).wait()

    stage_issue(0, 0)
    stage_issue(1, 1)
    stage_wait(0, 0)
    rows_issue(0, 0)

    nsuper = lax.div(e1w - ebase, CK * SUP) + 1

    def compute_chunk(c, rows_ref, dst_ref, ae_ref, carry):
        prev = carry[0]
        accs = list(carry[1:])
        base = ebase + c * CK
        dwin = dst_ref[...]
        for j in range(CK):
            d_j = dwin[j]
            neq = d_j != prev
            keepf = jnp.where(neq, 0.0, 1.0)

            @pl.when(neq)
            def _(accs=tuple(accs), prev=prev):
                m = jnp.clip(prev - n0, 0, NPW)
                for k in range(16):
                    out_v[m, pl.ds(16 * k, 16)] = accs[k]
                out_v[m, pl.ds(256, 16)] = accs[16]
                out_v[m, pl.ds(272, 16)] = accs[17]

            g = base + j
            wj = jnp.where((g >= e0) & (g < e1w), 1.0, 0.0)
            mj = jnp.clip(d_j - n0, 0, NPW)
            adv = ownad_v[mj, :]
            asv = rows_ref[j, pl.ds(HID + 16, 16)]
            aev = ae_ref[j, :]
            z = asv + adv + aev
            z = jnp.where(z >= 0.0, z, 0.2 * z)
            ex = jnp.exp(z) * wj
            naccs = [None] * 18
            naccs[16] = accs[16] * keepf + jnp.where(iota < 8, ex, 0.0)
            naccs[17] = accs[17] * keepf + aev * wj
            for k in range(16):
                naccs[k] = (accs[k] * keepf
                            + ex * rows_ref[j, pl.ds(16 * k, 16)])
            accs = naccs
            prev = d_j
        return (prev, *accs)

    def super_body(sit, carry):
        c0 = sit * SUP
        for k in range(SUP):
            c = c0 + k
            r2 = k % 2
            r3 = (k + 1) % 3
            rows_wait(r2, k % 3)
            stage_wait(c + 1, r3)
            rows_issue(1 - r2, r3)
            stage_issue(c + 2, (k + 2) % 3)
            carry = compute_chunk(c, rows[r2], dsts[k % 3], aes[k % 3], carry)
        return carry

    init = (jnp.int32(-1),) + tuple(jnp.zeros((16,), F32) for _ in range(18))
    lax.fori_loop(0, nsuper, super_body, init)

    pltpu.sync_copy(out_v.at[pl.ds(0, NPW)], out_hbm.at[pl.ds(n0, NPW)])


def _edge_sc_call(xa, ae_l, src_pad, dst_pad, rp_pad):
    mesh = plsc.VectorSubcoreMesh(core_axis_name="c", subcore_axis_name="s",
                                  num_cores=2, num_subcores=16)
    kfn = pl.kernel(
        _edge_sc_kernel,
        mesh=mesh,
        compiler_params=pltpu.CompilerParams(use_tc_tiling_on_sc=False,
                                             needs_layout_passes=False),
        out_type=jax.ShapeDtypeStruct((NP, XC), F32),
        scratch_types=[
            pltpu.VMEM((NPW + 1, 16), F32),
            pltpu.VMEM((16,), jnp.int32),
            pltpu.VMEM((CK,), jnp.int32),
            pltpu.VMEM((CK,), jnp.int32),
            pltpu.VMEM((CK,), jnp.int32),
            pltpu.VMEM((CK,), jnp.int32),
            pltpu.VMEM((CK,), jnp.int32),
            pltpu.VMEM((CK,), jnp.int32),
            pltpu.VMEM((CK, AEC), F32),
            pltpu.VMEM((CK, AEC), F32),
            pltpu.VMEM((CK, AEC), F32),
            pltpu.VMEM((CK, XC), F32),
            pltpu.VMEM((CK, XC), F32),
            pltpu.VMEM((NPW + 1, XC), F32),
            pltpu.SemaphoreType.DMA,
            pltpu.SemaphoreType.DMA,
            pltpu.SemaphoreType.DMA,
            pltpu.SemaphoreType.DMA,
            pltpu.SemaphoreType.DMA,
        ],
    )
    return kfn(xa, ae_l, src_pad, dst_pad, rp_pad)


PERM = (np.arange(HID) % NH) * C + np.arange(HID) // NH


def _fold_params(params):
    f = {}
    pe = params['edge_embed']
    ve = []
    for l in range(L):
        lp = params['layers'][l]
        ve.append((lp['W_e'].reshape(HID, NH, C) * lp['att_e'][None]).sum(-1))
    w2v = jnp.zeros((HID, L * AEC), F32)
    b2v = jnp.zeros((L * AEC,), F32)
    for l in range(L):
        pj = pe['W2'] @ ve[l]
        bj = pe['b2'] @ ve[l]
        w2v = w2v.at[:, l * AEC:l * AEC + NH].set(pj)
        w2v = w2v.at[:, l * AEC + NH:(l + 1) * AEC].set(pj)
        b2v = b2v.at[l * AEC:l * AEC + NH].set(bj)
        b2v = b2v.at[l * AEC + NH:(l + 1) * AEC].set(bj)
    f['w2v'] = w2v
    f['b2v'] = b2v
    wx = []
    lnp = []
    for l in range(L):
        lp = params['layers'][l]
        a_s = ((lp['W'].reshape(HID, NH, C)
                * lp['att_src'][None]).sum(-1))[PERM, :]
        a_d = ((lp['W'].reshape(HID, NH, C)
                * lp['att_dst'][None]).sum(-1))[PERM, :]
        wp = lp['W'][PERM, :][:, PERM]
        wx.append(jnp.concatenate(
            [wp, a_d, a_d, a_s, a_s], axis=1))
        lnp.append((lp['b'][PERM], lp['ln_g'][PERM], lp['ln_b'][PERM]))
    f['wx'] = wx
    f['lnp'] = lnp
    return f


@jax.jit
def kernel(x, edge_attr, params, edge_index, batch):
    src = edge_index[0]
    dst = edge_index[1]

    perm = jnp.argsort(dst)
    dst_s = dst[perm]
    src_s = src[perm]
    rp = jnp.searchsorted(dst_s, jnp.arange(N + 1, dtype=jnp.int32),
                          side='left').astype(jnp.int32)
    rp_pad = jnp.concatenate([rp, jnp.full((NP - N + RPPAD,), E, jnp.int32)])
    cntf = (jnp.maximum(jnp.diff(rp_pad[:NP + 1]).astype(F32), 0.0)[:, None]
            * jnp.ones((1, NH), F32))
    src_pad = jnp.concatenate([src_s, jnp.zeros((PADE,), jnp.int32)])
    dst_pad = jnp.concatenate([dst_s, jnp.full((PADE,), NP, jnp.int32)])

    fold = _fold_params(params)

    xp = jnp.zeros((NP, 16), F32).at[:N, :NF].set(x)
    pn = params['node_embed']
    w1p = jnp.concatenate([pn['W1'], jnp.zeros((16 - NF, HID), F32)], axis=0)
    h = _embed_call(xp, w1p, pn['b1'], pn['ln_g'], pn['ln_b'],
                    pn['W2'][:, PERM], pn['b2'][PERM], 2048, HID)

    ea_s = edge_attr[perm]
    eap = jnp.concatenate([ea_s, jnp.zeros((E, 8 - EF), F32)], axis=1)
    pe = params['edge_embed']
    w1e = jnp.concatenate([pe['W1'], jnp.zeros((8 - EF, HID), F32)], axis=0)
    ae_all = _embed_call(eap, w1e, pe['b1'], pe['ln_g'], pe['ln_b'],
                         fold['w2v'], fold['b2v'], 2000, L * AEC)
    ae_lay = ae_all.reshape(E, L, AEC).transpose(1, 0, 2)
    ae_lay = jnp.concatenate(
        [ae_lay, jnp.zeros((L, PADE, AEC), F32)], axis=1)

    for l in range(L):
        lp = params['layers'][l]
        xa = _mm_call(h, fold['wx'][l], 2048)
        acc = _edge_sc_call(xa, ae_lay[l], src_pad, dst_pad, rp_pad)
        hin = h if l > 0 else jnp.zeros((NP, HID), F32)
        bp, gp, bbp = fold['lnp'][l]
        h = _combine_call(acc, xa, cntf, hin, bp, gp, bbp, 2048)

    batch_pad = jnp.concatenate([batch, jnp.full((NP - N,), G, jnp.int32)])
    sums, maxs, cnts = _pool_call(h, batch_pad[:, None], 640)
    pp = np.concatenate([PERM, HID + PERM, 2 * HID + PERM])
    hp1 = dict(params['param_mlp'])
    hp2 = dict(params['metrics_mlp'])
    hp1['W1'] = hp1['W1'][pp, :]
    hp2['W1'] = hp2['W1'][pp, :]
    o1, o2 = _heads_call(sums, maxs, cnts, hp1, hp2)
    return o1, o2


# speedup vs baseline: 6.6592x; 6.6592x over previous
import functools

import jax
import jax.numpy as jnp
import numpy as np
from jax import lax
from jax.experimental import pallas as pl
from jax.experimental.pallas import tpu as pltpu
from jax.experimental.pallas import tpu_sc as plsc

N = 10000
NP = 10240
E = 160000
HID = 256
NH = 8
C = 32
G = 16
L = 8
NF = 10
EF = 4
XC = 288
AEC = 16
F32 = jnp.float32


def _ln(x, g, b):
    m = x.mean(-1, keepdims=True)
    v = ((x - m) ** 2).mean(-1, keepdims=True)
    return (x - m) / jnp.sqrt(v + 1e-5) * g + b


def _embed_body(x_ref, w1_ref, b1_ref, g_ref, bb_ref, w2_ref, b2_ref, o_ref):
    h = jnp.dot(x_ref[...], w1_ref[...], preferred_element_type=F32) + b1_ref[0]
    h = _ln(h, g_ref[0], bb_ref[0])
    h = jax.nn.gelu(h)
    o_ref[...] = jnp.dot(h, w2_ref[...], preferred_element_type=F32) + b2_ref[0]


def _mm_body(h_ref, w_ref, o_ref):
    o_ref[...] = jnp.dot(h_ref[...], w_ref[...], preferred_element_type=F32)


def _combine_body(acc_ref, xa_ref, hin_ref, b_ref, g_ref, bb_ref, o_ref):
    acc = acc_ref[...]
    xa = xa_ref[...]
    xh = xa[:, :HID]
    a_d = xa[:, HID:HID + NH]
    a_s = xa[:, HID + 16:HID + 16 + NH]
    den8 = acc[:, HID:HID + NH]
    aes8 = acc[:, HID + 2 * NH:HID + 3 * NH]
    cnt = acc[:, HID + 3 * NH:HID + 4 * NH]
    ael = aes8 / jnp.maximum(cnt, 1.0)
    z = a_s + a_d + ael
    z = jnp.where(z >= 0, z, 0.2 * z)
    exs = jnp.exp(z)
    heads = lax.broadcasted_iota(jnp.int32, (NH, HID), 1) % NH
    rows = lax.broadcasted_iota(jnp.int32, (NH, HID), 0)
    expm = jnp.where(heads == rows, 1.0, 0.0)
    num = acc[:, :HID] + jnp.dot(exs, expm, preferred_element_type=F32) * xh
    den = jnp.dot(den8 + exs, expm, preferred_element_type=F32)
    h2 = num / den + b_ref[0]
    h2 = jax.nn.gelu(_ln(h2, g_ref[0], bb_ref[0]))
    o_ref[...] = h2 + hin_ref[...]


def _pool_body(h_ref, b_ref, sum_ref, max_ref, cnt_ref):
    i = pl.program_id(0)

    @pl.when(i == 0)
    def _():
        sum_ref[...] = jnp.zeros_like(sum_ref)
        max_ref[...] = jnp.full_like(max_ref, -jnp.inf)
        cnt_ref[...] = jnp.zeros_like(cnt_ref)

    h = h_ref[...]
    bid = b_ref[...]
    bn = h.shape[0]
    gids = lax.broadcasted_iota(jnp.int32, (G, bn), 0)
    onehot = (gids == bid[:, 0][None, :]).astype(F32)
    sum_ref[...] += jnp.dot(onehot, h, preferred_element_type=F32)
    cnt_ref[...] += jnp.dot(onehot, jnp.ones((bn, NH), F32),
                            preferred_element_type=F32)
    rows = [jnp.max(jnp.where(bid == g, h, -jnp.inf), axis=0, keepdims=True)
            for g in range(G)]
    max_ref[...] = jnp.maximum(max_ref[...], jnp.concatenate(rows, axis=0))


def _heads_body(*refs):
    sum_ref, max_ref, cnt_ref = refs[0], refs[1], refs[2]
    wrefs = refs[3:-2]
    o1_ref, o2_ref = refs[-2], refs[-1]
    s = sum_ref[...]
    mx = max_ref[...]
    mx = jnp.where(mx > -1e37, mx, 0.0)
    c = jnp.maximum(cnt_ref[:, :1], 1.0)
    xg = jnp.concatenate([s / c, s, mx], axis=1)

    def head(ws):
        (w1, b1, g1, bb1, w2, b2, g2, bb2, w3, b3, g3, bb3, w4, b4) = ws
        h = xg @ w1[...] + b1[0]
        h = jax.nn.gelu(_ln(h, g1[0], bb1[0]))
        h = h @ w2[...] + b2[0]
        h = jax.nn.gelu(_ln(h, g2[0], bb2[0]))
        h = h @ w3[...] + b3[0]
        h = jax.nn.gelu(_ln(h, g3[0], bb3[0]))
        return jax.nn.sigmoid(h @ w4[...] + b4[0])

    o1_ref[...] = head(wrefs[:14])
    o2_ref[...] = head(wrefs[14:])


def _full(shape):
    return pl.BlockSpec(shape, lambda i: (0,) * len(shape))


def _embed_call(xp, w1, b1, g, bb, w2, b2, bn, out_cols):
    rows = xp.shape[0]
    k = xp.shape[1]
    grid = rows // bn
    return pl.pallas_call(
        _embed_body,
        grid=(grid,),
        in_specs=[
            pl.BlockSpec((bn, k), lambda i: (i, 0)),
            _full((k, HID)), _full((1, HID)), _full((1, HID)), _full((1, HID)),
            _full((HID, out_cols)), _full((1, out_cols)),
        ],
        out_specs=pl.BlockSpec((bn, out_cols), lambda i: (i, 0)),
        out_shape=jax.ShapeDtypeStruct((rows, out_cols), F32),
    )(xp, w1, b1[None], g[None], bb[None], w2, b2[None])


def _mm_call(h, w, bn):
    rows, k = h.shape
    cols = w.shape[1]
    return pl.pallas_call(
        _mm_body,
        grid=(rows // bn,),
        in_specs=[pl.BlockSpec((bn, k), lambda i: (i, 0)), _full((k, cols))],
        out_specs=pl.BlockSpec((bn, cols), lambda i: (i, 0)),
        out_shape=jax.ShapeDtypeStruct((rows, cols), F32),
    )(h, w)


def _combine_call(acc, xa, hin, b, g, bb, bn):
    grid = NP // bn
    return pl.pallas_call(
        _combine_body,
        grid=(grid,),
        in_specs=[
            pl.BlockSpec((bn, XC), lambda i: (i, 0)),
            pl.BlockSpec((bn, XC), lambda i: (i, 0)),
            pl.BlockSpec((bn, HID), lambda i: (i, 0)),
            _full((1, HID)), _full((1, HID)), _full((1, HID)),
        ],
        out_specs=pl.BlockSpec((bn, HID), lambda i: (i, 0)),
        out_shape=jax.ShapeDtypeStruct((NP, HID), F32),
    )(acc, xa, hin, b[None], g[None], bb[None])


def _pool_call(h, bid, bn):
    grid = NP // bn
    return pl.pallas_call(
        _pool_body,
        grid=(grid,),
        in_specs=[
            pl.BlockSpec((bn, HID), lambda i: (i, 0)),
            pl.BlockSpec((bn, 1), lambda i: (i, 0)),
        ],
        out_specs=[
            pl.BlockSpec((G, HID), lambda i: (0, 0)),
            pl.BlockSpec((G, HID), lambda i: (0, 0)),
            pl.BlockSpec((G, NH), lambda i: (0, 0)),
        ],
        out_shape=[
            jax.ShapeDtypeStruct((G, HID), F32),
            jax.ShapeDtypeStruct((G, HID), F32),
            jax.ShapeDtypeStruct((G, NH), F32),
        ],
    )(h, bid)


def _heads_call(sums, maxs, cnts, hp1, hp2):
    def wlist(p):
        out = []
        for wk, bk, gk, bbk in (('W1', 'b1', 'ln1_g', 'ln1_b'),
                                ('W2', 'b2', 'ln2_g', 'ln2_b'),
                                ('W3', 'b3', 'ln3_g', 'ln3_b')):
            out += [p[wk], p[bk][None], p[gk][None], p[bbk][None]]
        out += [p['W4'], p['b4'][None]]
        return out

    ws = wlist(hp1) + wlist(hp2)
    specs = [_full(w.shape) for w in ws]
    return pl.pallas_call(
        _heads_body,
        grid=(1,),
        in_specs=[_full((G, HID)), _full((G, HID)), _full((G, NH))] + specs,
        out_specs=[_full((G, 3)), _full((G, 3))],
        out_shape=[jax.ShapeDtypeStruct((G, 3), F32),
                   jax.ShapeDtypeStruct((G, 3), F32)],
    )(sums, maxs, cnts, *ws)


NPW = 320
RPPAD = NPW + 16
CK = 16
SUP = 6
PADE = 320


def _sc_wid():
    return lax.axis_index("s") * 2 + lax.axis_index("c")


def _edge_sc_kernel(xa_hbm, ae_hbm, src_hbm, dst_hbm, wb_hbm, out_hbm,
                    ownad_v, wb_v, s0, s1, s2, d0, d1, d2, a0, a1, a2,
                    r0, r1, out_v, st0, st1, st2, sr0, sr1):
    wid = _sc_wid()
    n0 = pl.multiple_of(wid * NPW, NPW)

    iota = lax.iota(jnp.int32, 16)
    zero16 = jnp.zeros((16,), F32)

    srcs = (s0, s1, s2)
    dsts = (d0, d1, d2)
    aes = (a0, a1, a2)
    rows = (r0, r1)
    stsems = (st0, st1, st2)
    rsems = (sr0, sr1)

    pltpu.sync_copy(wb_hbm, wb_v)
    e0 = jnp.int32(0)
    e1w = jnp.int32(0)
    for b in range(3):
        blk = wb_v[pl.ds(16 * b, 16)]
        e0 = e0 + jnp.sum(jnp.where(iota + 16 * b == wid, blk, 0))
        e1w = e1w + jnp.sum(jnp.where(iota + 16 * b == wid + 1, blk, 0))
    ebase = pl.multiple_of(e0 - lax.rem(e0, 8), 8)

    pltpu.sync_copy(xa_hbm.at[pl.ds(n0, NPW), pl.ds(HID, 16)],
                    ownad_v.at[pl.ds(0, NPW)])
    ownad_v[NPW, :] = zero16

    def zrow(m, _):
        for k in range(XC // 16):
            out_v[m, pl.ds(16 * k, 16)] = zero16
        return 0
    lax.fori_loop(0, NPW + 1, zrow, 0)

    def stage_issue(c, slot):
        off = ebase + c * CK
        pltpu.make_async_copy(src_hbm.at[pl.ds(off, CK)], srcs[slot],
                              stsems[slot]).start()
        pltpu.make_async_copy(dst_hbm.at[pl.ds(off, CK)], dsts[slot],
                              stsems[slot]).start()
        pltpu.make_async_copy(ae_hbm.at[pl.ds(off, CK), pl.ds(0, AEC)],
                              aes[slot], stsems[slot]).start()

    def stage_wait(c, slot):
        off = ebase + c * CK
        pltpu.make_async_copy(src_hbm.at[pl.ds(off, CK)], srcs[slot],
                              stsems[slot]).wait()
        pltpu.make_async_copy(dst_hbm.at[pl.ds(off, CK)], dsts[slot],
                              stsems[slot]).wait()
        pltpu.make_async_copy(ae_hbm.at[pl.ds(off, CK), pl.ds(0, AEC)],
                              aes[slot], stsems[slot]).wait()

    def rows_issue(rslot, sslot):
        pltpu.make_async_copy(xa_hbm.at[srcs[sslot]], rows[rslot],
                              rsems[rslot]).start()

    def rows_wait(rslot, sslot):
        pltpu.make_async_copy(xa_hbm.at[srcs[sslot]], rows[rslot],
                              rsems[rslot]).wait()

    stage_issue(0, 0)
    stage_issue(1, 1)
    stage_wait(0, 0)
    rows_issue(0, 0)

    nsuper = lax.div(e1w - ebase, CK * SUP) + 1

    def compute_chunk(c, rows_ref, dst_ref, ae_ref):
        base = ebase + c * CK
        dwin = dst_ref[...]

        def body_edge(j, _):
            d_j = jnp.sum(jnp.where(iota == j, dwin, 0))
            g = base + j
            wj = jnp.where((g >= e0) & (g < e1w), 1.0, 0.0)
            mj = jnp.clip(d_j - n0, 0, NPW)
            adv = ownad_v[mj, :]
            asv = rows_ref[j, pl.ds(HID + 16, 16)]
            aev = ae_ref[j, :]
            z = asv + adv + aev
            z = jnp.where(z >= 0.0, z, 0.2 * z)
            ex = jnp.exp(z) * wj
            for k in range(16):
                out_v[mj, pl.ds(16 * k, 16)] = (
                    out_v[mj, pl.ds(16 * k, 16)]
                    + ex * rows_ref[j, pl.ds(16 * k, 16)])
            out_v[mj, pl.ds(256, 16)] = out_v[mj, pl.ds(256, 16)] + ex
            tail = jnp.where(iota < 8, aev, 1.0) * wj
            out_v[mj, pl.ds(272, 16)] = out_v[mj, pl.ds(272, 16)] + tail
            return 0

        lax.fori_loop(0, CK, body_edge, 0)

    def super_body(sit, _):
        c0 = sit * SUP
        for k in range(SUP):
            c = c0 + k
            r2 = k % 2
            r3 = (k + 1) % 3
            rows_wait(r2, k % 3)
            stage_wait(c + 1, r3)
            rows_issue(1 - r2, r3)
            stage_issue(c + 2, (k + 2) % 3)
            compute_chunk(c, rows[r2], dsts[k % 3], aes[k % 3])
        return 0

    lax.fori_loop(0, nsuper, super_body, 0)

    rows_wait(0, 0)
    stage_wait(nsuper * SUP + 1, 1)

    pltpu.sync_copy(out_v.at[pl.ds(0, NPW)], out_hbm.at[pl.ds(n0, NPW)])


def _edge_sc_call(xa, ae_l, src_pad, dst_pad, wb_pad):
    mesh = plsc.VectorSubcoreMesh(core_axis_name="c", subcore_axis_name="s",
                                  num_cores=2, num_subcores=16)
    kfn = pl.kernel(
        _edge_sc_kernel,
        mesh=mesh,
        compiler_params=pltpu.CompilerParams(use_tc_tiling_on_sc=False,
                                             needs_layout_passes=False),
        out_type=jax.ShapeDtypeStruct((NP, XC), F32),
        scratch_types=[
            pltpu.VMEM((NPW + 1, 16), F32),
            pltpu.VMEM((48,), jnp.int32),
            pltpu.VMEM((CK,), jnp.int32),
            pltpu.VMEM((CK,), jnp.int32),
            pltpu.VMEM((CK,), jnp.int32),
            pltpu.VMEM((CK,), jnp.int32),
            pltpu.VMEM((CK,), jnp.int32),
            pltpu.VMEM((CK,), jnp.int32),
            pltpu.VMEM((CK, AEC), F32),
            pltpu.VMEM((CK, AEC), F32),
            pltpu.VMEM((CK, AEC), F32),
            pltpu.VMEM((CK, XC), F32),
            pltpu.VMEM((CK, XC), F32),
            pltpu.VMEM((NPW + 1, XC), F32),
            pltpu.SemaphoreType.DMA,
            pltpu.SemaphoreType.DMA,
            pltpu.SemaphoreType.DMA,
            pltpu.SemaphoreType.DMA,
            pltpu.SemaphoreType.DMA,
        ],
    )
    return kfn(xa, ae_l, src_pad, dst_pad, wb_pad)


PERM = (np.arange(HID) % NH) * C + np.arange(HID) // NH


def _fold_params(params):
    f = {}
    pe = params['edge_embed']
    ve = []
    for l in range(L):
        lp = params['layers'][l]
        ve.append((lp['W_e'].reshape(HID, NH, C) * lp['att_e'][None]).sum(-1))
    w2v = jnp.zeros((HID, L * AEC), F32)
    b2v = jnp.zeros((L * AEC,), F32)
    for l in range(L):
        pj = pe['W2'] @ ve[l]
        bj = pe['b2'] @ ve[l]
        w2v = w2v.at[:, l * AEC:l * AEC + NH].set(pj)
        w2v = w2v.at[:, l * AEC + NH:(l + 1) * AEC].set(pj)
        b2v = b2v.at[l * AEC:l * AEC + NH].set(bj)
        b2v = b2v.at[l * AEC + NH:(l + 1) * AEC].set(bj)
    f['w2v'] = w2v
    f['b2v'] = b2v
    wx = []
    lnp = []
    for l in range(L):
        lp = params['layers'][l]
        a_s = ((lp['W'].reshape(HID, NH, C)
                * lp['att_src'][None]).sum(-1))[PERM, :]
        a_d = ((lp['W'].reshape(HID, NH, C)
                * lp['att_dst'][None]).sum(-1))[PERM, :]
        wp = lp['W'][PERM, :][:, PERM]
        wx.append(jnp.concatenate(
            [wp, a_d, a_d, a_s, a_s], axis=1))
        lnp.append((lp['b'][PERM], lp['ln_g'][PERM], lp['ln_b'][PERM]))
    f['wx'] = wx
    f['lnp'] = lnp
    return f


@jax.jit
def kernel(x, edge_attr, params, edge_index, batch):
    src = edge_index[0]
    dst = edge_index[1]

    ops = lax.sort((dst, src, edge_attr[:, 0], edge_attr[:, 1],
                    edge_attr[:, 2], edge_attr[:, 3]), num_keys=1)
    dst_s, src_s = ops[0], ops[1]
    ea_s = jnp.stack(ops[2:6], axis=1)
    thr = jnp.arange(0, NP + 1, NPW, dtype=jnp.int32)
    wb = jnp.sum((dst_s[None, :] < thr[:, None]).astype(jnp.int32), axis=1)
    wb_pad = jnp.concatenate([wb, jnp.full((15,), E, jnp.int32)])
    src_pad = jnp.concatenate([src_s, jnp.zeros((PADE,), jnp.int32)])
    dst_pad = jnp.concatenate([dst_s, jnp.full((PADE,), NP, jnp.int32)])

    fold = _fold_params(params)

    xp = jnp.zeros((NP, 16), F32).at[:N, :NF].set(x)
    pn = params['node_embed']
    w1p = jnp.concatenate([pn['W1'], jnp.zeros((16 - NF, HID), F32)], axis=0)
    h = _embed_call(xp, w1p, pn['b1'], pn['ln_g'], pn['ln_b'],
                    pn['W2'][:, PERM], pn['b2'][PERM], 2048, HID)

    eap = jnp.concatenate([ea_s, jnp.zeros((E, 8 - EF), F32)], axis=1)
    pe = params['edge_embed']
    w1e = jnp.concatenate([pe['W1'], jnp.zeros((8 - EF, HID), F32)], axis=0)
    ae_all = _embed_call(eap, w1e, pe['b1'], pe['ln_g'], pe['ln_b'],
                         fold['w2v'], fold['b2v'], 2000, L * AEC)
    ae_lay = ae_all.reshape(E, L, AEC).transpose(1, 0, 2)
    ae_lay = jnp.concatenate(
        [ae_lay, jnp.zeros((L, PADE, AEC), F32)], axis=1)

    for l in range(L):
        lp = params['layers'][l]
        xa = _mm_call(h, fold['wx'][l], 2048)
        acc = _edge_sc_call(xa, ae_lay[l], src_pad, dst_pad, wb_pad)
        hin = h if l > 0 else jnp.zeros((NP, HID), F32)
        bp, gp, bbp = fold['lnp'][l]
        h = _combine_call(acc, xa, hin, bp, gp, bbp, 2048)

    batch_pad = jnp.concatenate([batch, jnp.full((NP - N,), G, jnp.int32)])
    sums, maxs, cnts = _pool_call(h, batch_pad[:, None], 640)
    pp = np.concatenate([PERM, HID + PERM, 2 * HID + PERM])
    hp1 = dict(params['param_mlp'])
    hp2 = dict(params['metrics_mlp'])
    hp1['W1'] = hp1['W1'][pp, :]
    hp2['W1'] = hp2['W1'][pp, :]
    o1, o2 = _heads_call(sums, maxs, cnts, hp1, hp2)
    return o1, o2
